# baseline (speedup 1.0000x reference)
import jax
import jax.numpy as jnp
from jax import lax
from jax.experimental import pallas as pl
from jax.experimental.pallas import tpu as pltpu
from jax.experimental.pallas import tpu_sc as plsc

N = 10000
E = 320000
D = 128
H = D // 2

NC = 2
NS = 16
BCH = 125
KCH = E // NS // BCH
NB = 4
FK = 10
NPAD = 10112
RPS = NPAD // NS
RCH = (160, 160, 160, 152)

_mesh = plsc.VectorSubcoreMesh(core_axis_name="c", subcore_axis_name="s")

KCH1 = E // (NC * NS) // BCH


def _deg_body(dst_hbm, ones_hbm, zeros_hbm, deg_out, dst_v, ones_v, deg_sh, sem):
    c = lax.axis_index("c")
    s = lax.axis_index("s")
    wid = s * NC + c
    pltpu.sync_copy(dst_hbm.at[wid], dst_v)
    pltpu.sync_copy(ones_hbm, ones_v)
    r0 = s * RPS
    pltpu.sync_copy(zeros_hbm.at[pl.ds(r0, RPS)], deg_sh.at[pl.ds(r0, RPS)])
    plsc.subcore_barrier()

    def body(i, carry):
        for b in range(FK):
            pltpu.async_copy(ones_v, deg_sh.at[dst_v.at[i * FK + b]], sem, add=True)
        for b in range(FK):
            pltpu.make_async_copy(ones_v, deg_sh.at[dst_v.at[i * FK + b]], sem).wait()
        return carry

    lax.fori_loop(0, KCH1 // FK, body, 0)
    plsc.subcore_barrier()
    pltpu.sync_copy(deg_sh.at[pl.ds(r0, RPS)], deg_out.at[c, pl.ds(r0, RPS)])


def _deg_types(interpret=False):
    return dict(
        out_type=jax.ShapeDtypeStruct((NC, NPAD, 16), jnp.float32),
        mesh=_mesh,
        scratch_types=[
            pltpu.VMEM((KCH1, BCH), jnp.int32),
            pltpu.VMEM((BCH, 16), jnp.float32),
            pltpu.VMEM_SHARED((NPAD, 16), jnp.float32),
            pltpu.SemaphoreType.DMA,
        ],
        compiler_params=pltpu.CompilerParams(use_tc_tiling_on_sc=False),
        interpret=interpret,
    )


_deg_kernel = pl.kernel(_deg_body, **_deg_types())


def _mega_body(src_hbm, dst_hbm, x2_hbm, deg2_hbm, zeros64_hbm,
               xs2_out, agg_out,
               src_v, dst_v, rows_v, xbuf, degbuf0, degbuf1,
               agg_sh, sg, ss):
    c = lax.axis_index("c")
    s = lax.axis_index("s")
    r0 = s * RPS

    pltpu.sync_copy(src_hbm.at[s], src_v)
    pltpu.sync_copy(dst_hbm.at[s], dst_v)
    pltpu.sync_copy(zeros64_hbm.at[pl.ds(r0, RPS)], agg_sh.at[pl.ds(r0, RPS)])
    plsc.subcore_barrier()

    a = 0
    for sz in RCH:
        row0 = r0 + a
        pltpu.sync_copy(x2_hbm.at[c, pl.ds(row0, sz)], xbuf.at[pl.ds(0, sz)])
        pltpu.sync_copy(deg2_hbm.at[0, pl.ds(row0, sz)], degbuf0.at[pl.ds(0, sz)])
        pltpu.sync_copy(deg2_hbm.at[1, pl.ds(row0, sz)], degbuf1.at[pl.ds(0, sz)])

        def row_body(r, carry):
            dv = degbuf0[r] + degbuf1[r] + 1.0
            hv = dv * 0.5
            iv = jnp.int32(0x5F3759DF) - lax.shift_right_logical(
                plsc.bitcast(dv, jnp.int32), 1)
            y = plsc.bitcast(iv, jnp.float32)
            y = y * (1.5 - hv * y * y)
            y = y * (1.5 - hv * y * y)
            y = y * (1.5 - hv * y * y)
            for q in range(H // 16):
                xbuf[r, pl.ds(16 * q, 16)] = xbuf[r, pl.ds(16 * q, 16)] * y
            return carry

        lax.fori_loop(0, sz, row_body, 0)
        pltpu.sync_copy(xbuf.at[pl.ds(0, sz)], xs2_out.at[c, pl.ds(row0, sz)])
        a += sz
    plsc.subcore_barrier()

    xs_c = xs2_out.at[c]
    for b in range(NB):
        pltpu.async_copy(xs_c.at[src_v.at[b]], rows_v.at[b], sg[b])

    def edge_body(i, carry):
        for b in range(NB):
            j = i * NB + b
            pltpu.make_async_copy(xs_c.at[src_v.at[j]], rows_v.at[b], sg[b]).wait()
            pltpu.async_copy(rows_v.at[b], agg_sh.at[dst_v.at[j]], ss[b], add=True)
            nxt = j + NB

            @pl.when(nxt < KCH)
            def _():
                pltpu.make_async_copy(rows_v.at[b], agg_sh.at[dst_v.at[j]], ss[b]).wait()
                pltpu.async_copy(xs_c.at[src_v.at[nxt]], rows_v.at[b], sg[b])
        return carry

    lax.fori_loop(0, KCH // NB, edge_body, 0)
    for b in range(NB):
        pltpu.make_async_copy(rows_v.at[b], agg_sh.at[dst_v.at[KCH - NB + b]], ss[b]).wait()
    plsc.subcore_barrier()

    pltpu.sync_copy(agg_sh.at[pl.ds(r0, RPS)], agg_out.at[c, pl.ds(r0, RPS)])


def _mega_types(interpret=False):
    return dict(
        out_type=(
            jax.ShapeDtypeStruct((NC, NPAD, H), jnp.float32),
            jax.ShapeDtypeStruct((NC, NPAD, H), jnp.float32),
        ),
        mesh=_mesh,
        scratch_types=[
            pltpu.VMEM((KCH, BCH), jnp.int32),
            pltpu.VMEM((KCH, BCH), jnp.int32),
            pltpu.VMEM((NB, BCH, H), jnp.float32),
            pltpu.VMEM((RCH[0], H), jnp.float32),
            pltpu.VMEM((RCH[0], 16), jnp.float32),
            pltpu.VMEM((RCH[0], 16), jnp.float32),
            pltpu.VMEM_SHARED((NPAD, H), jnp.float32),
            [pltpu.SemaphoreType.DMA] * NB,
            [pltpu.SemaphoreType.DMA] * NB,
        ],
        compiler_params=pltpu.CompilerParams(
            use_tc_tiling_on_sc=False, needs_layout_passes=False),
        interpret=interpret,
    )


_mega_kernel = pl.kernel(_mega_body, **_mega_types())


_BLK = 1000


def _out_body(a_ref, xs_ref, deg_ref, wm_ref, wl_ref, bm_ref, bl_ref,
              mu_ref, ls_ref):
    deg = (deg_ref[0] + deg_ref[1])[:, 0:1] + 1.0
    dis = jnp.broadcast_to(lax.rsqrt(deg), (_BLK, D))
    agg = jnp.concatenate([a_ref[0] + xs_ref[0], a_ref[1] + xs_ref[1]], axis=1)
    full = dis * agg
    mu_ref[...] = (
        jnp.dot(full, wm_ref[...], preferred_element_type=jnp.float32)
        + bm_ref[0:1, :]
    )
    ls_ref[...] = (
        jnp.dot(full, wl_ref[...], preferred_element_type=jnp.float32)
        + bl_ref[0:1, :]
    )


def _out_kernel(agg2, xs2, deg2, W_mu, W_ls, bm8, bl8):
    halves = pl.BlockSpec((NC, _BLK, H), lambda i: (0, i, 0))
    deg = pl.BlockSpec((NC, _BLK, 16), lambda i: (0, i, 0))
    row = pl.BlockSpec((_BLK, D), lambda i: (i, 0))
    whole = pl.BlockSpec((D, D), lambda i: (0, 0))
    bias = pl.BlockSpec((8, D), lambda i: (0, 0))
    return pl.pallas_call(
        _out_body,
        grid=(N // _BLK,),
        in_specs=[halves, halves, deg, whole, whole, bias, bias],
        out_specs=[row, row],
        out_shape=[
            jax.ShapeDtypeStruct((N, D), jnp.float32),
            jax.ShapeDtypeStruct((N, D), jnp.float32),
        ],
    )(agg2, xs2, deg2, W_mu, W_ls, bm8, bl8)


def kernel(x, edge_index, W_mu, b_mu, W_logstd, b_logstd):
    src_r = edge_index[0].reshape(NS, KCH, BCH)
    dst_r1 = edge_index[1].reshape(NC * NS, KCH1, BCH)
    dst_r = edge_index[1].reshape(NS, KCH, BCH)
    x2p = jnp.zeros((NC, NPAD, H), jnp.float32)
    x2p = x2p.at[0, :N].set(x[:, :H]).at[1, :N].set(x[:, H:])
    ones16 = jnp.ones((BCH, 16), jnp.float32)
    zeros16 = jnp.zeros((NPAD, 16), jnp.float32)
    zeros64 = jnp.zeros((NPAD, H), jnp.float32)

    deg2 = _deg_kernel(dst_r1, ones16, zeros16)
    xs2, agg2 = _mega_kernel(src_r, dst_r, x2p, deg2, zeros64)
    mu, logstd = _out_kernel(
        agg2, xs2, deg2, W_mu, W_logstd,
        jnp.broadcast_to(b_mu[None, :], (8, D)),
        jnp.broadcast_to(b_logstd[None, :], (8, D)),
    )
    return (mu, logstd)


# speedup vs baseline: 1.0271x; 1.0271x over previous
import jax
import jax.numpy as jnp
from jax import lax
from jax.experimental import pallas as pl
from jax.experimental.pallas import tpu as pltpu
from jax.experimental.pallas import tpu_sc as plsc

N = 10000
E = 320000
D = 128
H = D // 2

NC = 2
NS = 16
BCH = 125
KCH = E // NS // BCH
NB = 4
FK = 10
NPAD = 10112
RPS = NPAD // NS
RCH = (160, 160, 160, 152)

_mesh = plsc.VectorSubcoreMesh(core_axis_name="c", subcore_axis_name="s")

KCH1 = E // (NC * NS) // BCH


def _deg_body(dst_hbm, ones_hbm, zeros_hbm, deg_out, dst_v, ones_v, deg_sh, sem):
    c = lax.axis_index("c")
    s = lax.axis_index("s")
    wid = s * NC + c
    pltpu.sync_copy(dst_hbm.at[wid], dst_v)
    pltpu.sync_copy(ones_hbm, ones_v)
    r0 = s * RPS
    pltpu.sync_copy(zeros_hbm.at[pl.ds(r0, RPS)], deg_sh.at[pl.ds(r0, RPS)])
    plsc.subcore_barrier()

    def body(i, carry):
        for b in range(FK):
            pltpu.async_copy(ones_v, deg_sh.at[dst_v.at[i * FK + b]], sem, add=True)
        for b in range(FK):
            pltpu.make_async_copy(ones_v, deg_sh.at[dst_v.at[i * FK + b]], sem).wait()
        return carry

    lax.fori_loop(0, KCH1 // FK, body, 0)
    plsc.subcore_barrier()
    pltpu.sync_copy(deg_sh.at[pl.ds(r0, RPS)], deg_out.at[c, pl.ds(r0, RPS)])


def _deg_types(interpret=False):
    return dict(
        out_type=jax.ShapeDtypeStruct((NC, NPAD, 16), jnp.float32),
        mesh=_mesh,
        scratch_types=[
            pltpu.VMEM((KCH1, BCH), jnp.int32),
            pltpu.VMEM((BCH, 16), jnp.float32),
            pltpu.VMEM_SHARED((NPAD, 16), jnp.float32),
            pltpu.SemaphoreType.DMA,
        ],
        compiler_params=pltpu.CompilerParams(use_tc_tiling_on_sc=False),
        interpret=interpret,
    )


_deg_kernel = pl.kernel(_deg_body, **_deg_types())


def _mega_body(src_hbm, dst_hbm, x2_hbm, deg2_hbm, zeros64_hbm,
               xs2_out, agg_out,
               src_v, dst_v, rows_v, xbuf, degbuf0, degbuf1,
               agg_sh, sg, ss):
    c = lax.axis_index("c")
    s = lax.axis_index("s")
    r0 = s * RPS

    pltpu.sync_copy(src_hbm.at[s], src_v)
    pltpu.sync_copy(dst_hbm.at[s], dst_v)
    pltpu.sync_copy(zeros64_hbm.at[pl.ds(r0, RPS)], agg_sh.at[pl.ds(r0, RPS)])
    plsc.subcore_barrier()

    def _nrsqrt(dv):
        hv = dv * 0.5
        iv = jnp.int32(0x5F3759DF) - lax.shift_right_logical(
            plsc.bitcast(dv, jnp.int32), 1)
        y = plsc.bitcast(iv, jnp.float32)
        y = y * (1.5 - hv * y * y)
        y = y * (1.5 - hv * y * y)
        return y

    a = 0
    for sz in RCH:
        row0 = r0 + a
        pltpu.sync_copy(x2_hbm.at[c, pl.ds(row0, sz)], xbuf.at[pl.ds(0, sz)])
        pltpu.sync_copy(deg2_hbm.at[0, pl.ds(row0, sz)], degbuf0.at[pl.ds(0, sz)])
        pltpu.sync_copy(deg2_hbm.at[1, pl.ds(row0, sz)], degbuf1.at[pl.ds(0, sz)])

        def dis_body(i, carry):
            for u in range(2):
                r = i * 2 + u
                degbuf0[r] = _nrsqrt(degbuf0[r] + degbuf1[r] + 1.0)
            return carry

        lax.fori_loop(0, sz // 2, dis_body, 0)

        def mul_body(i, carry):
            for u in range(2):
                r = i * 2 + u
                y = degbuf0[r]
                for q in range(H // 16):
                    xbuf[r, pl.ds(16 * q, 16)] = xbuf[r, pl.ds(16 * q, 16)] * y
            return carry

        lax.fori_loop(0, sz // 2, mul_body, 0)
        pltpu.sync_copy(xbuf.at[pl.ds(0, sz)], xs2_out.at[c, pl.ds(row0, sz)])
        a += sz
    plsc.subcore_barrier()

    xs_c = xs2_out.at[c]
    for b in range(NB):
        pltpu.async_copy(xs_c.at[src_v.at[b]], rows_v.at[b], sg[b])

    def edge_body(i, carry):
        for b in range(NB):
            j = i * NB + b
            pltpu.make_async_copy(xs_c.at[src_v.at[j]], rows_v.at[b], sg[b]).wait()
            pltpu.async_copy(rows_v.at[b], agg_sh.at[dst_v.at[j]], ss[b], add=True)
            nxt = j + NB

            @pl.when(nxt < KCH)
            def _():
                pltpu.make_async_copy(rows_v.at[b], agg_sh.at[dst_v.at[j]], ss[b]).wait()
                pltpu.async_copy(xs_c.at[src_v.at[nxt]], rows_v.at[b], sg[b])
        return carry

    lax.fori_loop(0, KCH // NB, edge_body, 0)
    for b in range(NB):
        pltpu.make_async_copy(rows_v.at[b], agg_sh.at[dst_v.at[KCH - NB + b]], ss[b]).wait()
    plsc.subcore_barrier()

    pltpu.sync_copy(agg_sh.at[pl.ds(r0, RPS)], agg_out.at[c, pl.ds(r0, RPS)])


def _mega_types(interpret=False):
    return dict(
        out_type=(
            jax.ShapeDtypeStruct((NC, NPAD, H), jnp.float32),
            jax.ShapeDtypeStruct((NC, NPAD, H), jnp.float32),
        ),
        mesh=_mesh,
        scratch_types=[
            pltpu.VMEM((KCH, BCH), jnp.int32),
            pltpu.VMEM((KCH, BCH), jnp.int32),
            pltpu.VMEM((NB, BCH, H), jnp.float32),
            pltpu.VMEM((RCH[0], H), jnp.float32),
            pltpu.VMEM((RCH[0], 16), jnp.float32),
            pltpu.VMEM((RCH[0], 16), jnp.float32),
            pltpu.VMEM_SHARED((NPAD, H), jnp.float32),
            [pltpu.SemaphoreType.DMA] * NB,
            [pltpu.SemaphoreType.DMA] * NB,
        ],
        compiler_params=pltpu.CompilerParams(
            use_tc_tiling_on_sc=False, needs_layout_passes=False),
        interpret=interpret,
    )


_mega_kernel = pl.kernel(_mega_body, **_mega_types())


_BLK = 1000


def _out_body(a_ref, xs_ref, deg_ref, wm_ref, wl_ref, bm_ref, bl_ref,
              mu_ref, ls_ref):
    deg = (deg_ref[0] + deg_ref[1])[:, 0:1] + 1.0
    dis = jnp.broadcast_to(lax.rsqrt(deg), (_BLK, D))
    agg = jnp.concatenate([a_ref[0] + xs_ref[0], a_ref[1] + xs_ref[1]], axis=1)
    full = dis * agg
    mu_ref[...] = (
        jnp.dot(full, wm_ref[...], preferred_element_type=jnp.float32)
        + bm_ref[0:1, :]
    )
    ls_ref[...] = (
        jnp.dot(full, wl_ref[...], preferred_element_type=jnp.float32)
        + bl_ref[0:1, :]
    )


def _out_kernel(agg2, xs2, deg2, W_mu, W_ls, bm8, bl8):
    halves = pl.BlockSpec((NC, _BLK, H), lambda i: (0, i, 0))
    deg = pl.BlockSpec((NC, _BLK, 16), lambda i: (0, i, 0))
    row = pl.BlockSpec((_BLK, D), lambda i: (i, 0))
    whole = pl.BlockSpec((D, D), lambda i: (0, 0))
    bias = pl.BlockSpec((8, D), lambda i: (0, 0))
    return pl.pallas_call(
        _out_body,
        grid=(N // _BLK,),
        in_specs=[halves, halves, deg, whole, whole, bias, bias],
        out_specs=[row, row],
        out_shape=[
            jax.ShapeDtypeStruct((N, D), jnp.float32),
            jax.ShapeDtypeStruct((N, D), jnp.float32),
        ],
    )(agg2, xs2, deg2, W_mu, W_ls, bm8, bl8)


def kernel(x, edge_index, W_mu, b_mu, W_logstd, b_logstd):
    src_r = edge_index[0].reshape(NS, KCH, BCH)
    dst_r1 = edge_index[1].reshape(NC * NS, KCH1, BCH)
    dst_r = edge_index[1].reshape(NS, KCH, BCH)
    x2p = jnp.zeros((NC, NPAD, H), jnp.float32)
    x2p = x2p.at[0, :N].set(x[:, :H]).at[1, :N].set(x[:, H:])
    ones16 = jnp.ones((BCH, 16), jnp.float32)
    zeros16 = jnp.zeros((NPAD, 16), jnp.float32)
    zeros64 = jnp.zeros((NPAD, H), jnp.float32)

    deg2 = _deg_kernel(dst_r1, ones16, zeros16)
    xs2, agg2 = _mega_kernel(src_r, dst_r, x2p, deg2, zeros64)
    mu, logstd = _out_kernel(
        agg2, xs2, deg2, W_mu, W_logstd,
        jnp.broadcast_to(b_mu[None, :], (8, D)),
        jnp.broadcast_to(b_logstd[None, :], (8, D)),
    )
    return (mu, logstd)


# speedup vs baseline: 1.1331x; 1.1032x over previous
import functools

import jax
import jax.numpy as jnp
from jax import lax
from jax.experimental import pallas as pl
from jax.experimental.pallas import tpu as pltpu
from jax.experimental.pallas import tpu_sc as plsc

N = 10000
E = 320000
D = 128
H = D // 2

NC = 2
NS = 16
NW = NC * NS
BCH = 125
KCH1 = E // NW // BCH
KCH2 = E // NS // BCH
NB = 4
NPAD = 10112
RPS = NPAD // NS

_mesh = plsc.VectorSubcoreMesh(core_axis_name="c", subcore_axis_name="s")


def _deg_body(dst_hbm, ones_hbm, zeros_hbm, deg_out, dst_v, ones_v, deg_sh, sem):
    c = lax.axis_index("c")
    s = lax.axis_index("s")
    wid = s * NC + c
    pltpu.sync_copy(dst_hbm.at[wid], dst_v)
    pltpu.sync_copy(ones_hbm, ones_v)
    r0 = s * RPS
    pltpu.sync_copy(zeros_hbm.at[pl.ds(r0, RPS)], deg_sh.at[pl.ds(r0, RPS)])
    plsc.subcore_barrier()

    FK = 10

    def body(i, carry):
        for b in range(FK):
            pltpu.async_copy(ones_v, deg_sh.at[dst_v.at[i * FK + b]], sem, add=True)
        for b in range(FK):
            pltpu.make_async_copy(ones_v, deg_sh.at[dst_v.at[i * FK + b]], sem).wait()
        return carry

    lax.fori_loop(0, KCH1 // FK, body, 0)
    plsc.subcore_barrier()
    pltpu.sync_copy(deg_sh.at[pl.ds(r0, RPS)], deg_out.at[c, pl.ds(r0, RPS)])


def _deg_types(interpret=False):
    return dict(
        out_type=jax.ShapeDtypeStruct((NC, NPAD, 16), jnp.float32),
        mesh=_mesh,
        scratch_types=[
            pltpu.VMEM((KCH1, BCH), jnp.int32),
            pltpu.VMEM((BCH, 16), jnp.float32),
            pltpu.VMEM_SHARED((NPAD, 16), jnp.float32),
            pltpu.SemaphoreType.DMA,
        ],
        compiler_params=pltpu.CompilerParams(use_tc_tiling_on_sc=False),
        interpret=interpret,
    )


_deg_kernel = pl.kernel(_deg_body, **_deg_types())


def _agg_body(src_hbm, dst_hbm, xs_hbm, zeros_hbm, agg_out,
              src_v, dst_v, rows_v, agg_sh, sg, ss):
    c = lax.axis_index("c")
    s = lax.axis_index("s")
    pltpu.sync_copy(src_hbm.at[s], src_v)
    pltpu.sync_copy(dst_hbm.at[s], dst_v)
    r0 = s * RPS
    pltpu.sync_copy(zeros_hbm.at[pl.ds(r0, RPS)], agg_sh.at[pl.ds(r0, RPS)])
    plsc.subcore_barrier()

    xs_c = xs_hbm.at[c]
    for b in range(NB):
        pltpu.async_copy(xs_c.at[src_v.at[b]], rows_v.at[b], sg[b])

    def body(i, carry):
        for b in range(NB):
            j = i * NB + b
            pltpu.make_async_copy(xs_c.at[src_v.at[j]], rows_v.at[b], sg[b]).wait()
            pltpu.async_copy(rows_v.at[b], agg_sh.at[dst_v.at[j]], ss[b], add=True)
            nxt = j + NB

            @pl.when(nxt < KCH2)
            def _():
                pltpu.make_async_copy(rows_v.at[b], agg_sh.at[dst_v.at[j]], ss[b]).wait()
                pltpu.async_copy(xs_c.at[src_v.at[nxt]], rows_v.at[b], sg[b])
        return carry

    lax.fori_loop(0, KCH2 // NB, body, 0)
    for b in range(NB):
        pltpu.make_async_copy(rows_v.at[b], agg_sh.at[dst_v.at[KCH2 - NB + b]], ss[b]).wait()
    plsc.subcore_barrier()
    pltpu.sync_copy(agg_sh.at[pl.ds(r0, RPS)], agg_out.at[c, pl.ds(r0, RPS)])


def _agg_types(interpret=False):
    return dict(
        out_type=jax.ShapeDtypeStruct((NC, NPAD, H), jnp.float32),
        mesh=_mesh,
        scratch_types=[
            pltpu.VMEM((KCH2, BCH), jnp.int32),
            pltpu.VMEM((KCH2, BCH), jnp.int32),
            pltpu.VMEM((NB, BCH, H), jnp.float32),
            pltpu.VMEM_SHARED((NPAD, H), jnp.float32),
            [pltpu.SemaphoreType.DMA] * NB,
            [pltpu.SemaphoreType.DMA] * NB,
        ],
        compiler_params=pltpu.CompilerParams(use_tc_tiling_on_sc=False),
        interpret=interpret,
    )


_agg_kernel = pl.kernel(_agg_body, **_agg_types())


_BLK = 1000


def _dis_block(d_ref):
    deg = (d_ref[0] + d_ref[1])[:, 0:1] + 1.0
    return jnp.broadcast_to(lax.rsqrt(deg), (_BLK, D))


def _scale_body(d_ref, x_ref, xs_ref):
    xs = x_ref[...] * _dis_block(d_ref)
    xs_ref[0] = xs[:, :H]
    xs_ref[1] = xs[:, H:]


def _scale_kernel(deg2, x):
    return pl.pallas_call(
        _scale_body,
        grid=(N // _BLK,),
        in_specs=[
            pl.BlockSpec((NC, _BLK, 16), lambda i: (0, i, 0)),
            pl.BlockSpec((_BLK, D), lambda i: (i, 0)),
        ],
        out_specs=pl.BlockSpec((NC, _BLK, H), lambda i: (0, i, 0)),
        out_shape=jax.ShapeDtypeStruct((NC, N, H), jnp.float32),
    )(deg2, x)


def _out_body(a_ref, xs_ref, d_ref, wm_ref, wl_ref,
              bm_ref, bl_ref, mu_ref, ls_ref):
    agg = jnp.concatenate([a_ref[0] + xs_ref[0], a_ref[1] + xs_ref[1]], axis=1)
    full = _dis_block(d_ref) * agg
    mu_ref[...] = (
        jnp.dot(full, wm_ref[...], preferred_element_type=jnp.float32)
        + bm_ref[0:1, :]
    )
    ls_ref[...] = (
        jnp.dot(full, wl_ref[...], preferred_element_type=jnp.float32)
        + bl_ref[0:1, :]
    )


def _out_kernel(agg2, xs2, deg2, W_mu, W_ls, bm8, bl8):
    halves = pl.BlockSpec((NC, _BLK, H), lambda i: (0, i, 0))
    deg = pl.BlockSpec((NC, _BLK, 16), lambda i: (0, i, 0))
    row = pl.BlockSpec((_BLK, D), lambda i: (i, 0))
    whole = pl.BlockSpec((D, D), lambda i: (0, 0))
    bias = pl.BlockSpec((8, D), lambda i: (0, 0))
    return pl.pallas_call(
        _out_body,
        grid=(N // _BLK,),
        in_specs=[halves, halves, deg, whole, whole, bias, bias],
        out_specs=[row, row],
        out_shape=[
            jax.ShapeDtypeStruct((N, D), jnp.float32),
            jax.ShapeDtypeStruct((N, D), jnp.float32),
        ],
    )(agg2, xs2, deg2, W_mu, W_ls, bm8, bl8)


def kernel(x, edge_index, W_mu, b_mu, W_logstd, b_logstd):
    dst_r1 = edge_index[1].reshape(NW, KCH1, BCH)
    src_r2 = edge_index[0].reshape(NS, KCH2, BCH)
    dst_r2 = edge_index[1].reshape(NS, KCH2, BCH)
    ones16 = jnp.ones((BCH, 16), jnp.float32)
    zeros16 = jnp.zeros((NPAD, 16), jnp.float32)
    zeros64 = jnp.zeros((NPAD, H), jnp.float32)

    deg2 = _deg_kernel(dst_r1, ones16, zeros16)
    xs2 = _scale_kernel(deg2, x)
    agg2 = _agg_kernel(src_r2, dst_r2, xs2, zeros64)
    mu, logstd = _out_kernel(
        agg2, xs2, deg2, W_mu, W_logstd,
        jnp.broadcast_to(b_mu[None, :], (8, D)),
        jnp.broadcast_to(b_logstd[None, :], (8, D)),
    )
    return (mu, logstd)


# speedup vs baseline: 1.1523x; 1.0169x over previous
import functools

import jax
import jax.numpy as jnp
from jax import lax
from jax.experimental import pallas as pl
from jax.experimental.pallas import tpu as pltpu
from jax.experimental.pallas import tpu_sc as plsc

N = 10000
E = 320000
D = 128
H = D // 2

NC = 2
NS = 16
NW = NC * NS
BCH = 125
KCH1 = E // NW // BCH
KCH2 = E // NS // BCH
NB = 4
NPAD = 10112
RPS = NPAD // NS

_mesh = plsc.VectorSubcoreMesh(core_axis_name="c", subcore_axis_name="s")


def _deg_body(dst_hbm, ones_hbm, zeros_hbm, deg_out, dst_v, ones_v, deg_sh, sem):
    c = lax.axis_index("c")
    s = lax.axis_index("s")
    wid = s * NC + c
    pltpu.sync_copy(dst_hbm.at[wid], dst_v)
    pltpu.sync_copy(ones_hbm, ones_v)
    r0 = s * RPS
    pltpu.sync_copy(zeros_hbm.at[pl.ds(r0, RPS)], deg_sh.at[pl.ds(r0, RPS)])
    plsc.subcore_barrier()

    FK = 10

    def body(i, carry):
        for b in range(FK):
            pltpu.async_copy(ones_v, deg_sh.at[dst_v.at[i * FK + b]], sem, add=True)
        for b in range(FK):
            pltpu.make_async_copy(ones_v, deg_sh.at[dst_v.at[i * FK + b]], sem).wait()
        return carry

    lax.fori_loop(0, KCH1 // FK, body, 0)
    plsc.subcore_barrier()
    pltpu.sync_copy(deg_sh.at[pl.ds(r0, RPS)], deg_out.at[c, pl.ds(r0, RPS)])


def _deg_types(interpret=False):
    return dict(
        out_type=jax.ShapeDtypeStruct((NC, NPAD, 16), jnp.float32),
        mesh=_mesh,
        scratch_types=[
            pltpu.VMEM((KCH1, BCH), jnp.int32),
            pltpu.VMEM((BCH, 16), jnp.float32),
            pltpu.VMEM_SHARED((NPAD, 16), jnp.float32),
            pltpu.SemaphoreType.DMA,
        ],
        compiler_params=pltpu.CompilerParams(use_tc_tiling_on_sc=False),
        interpret=interpret,
    )


_deg_kernel = pl.kernel(_deg_body, **_deg_types())


def _agg_body(src_hbm, dst_hbm, xs_hbm, zeros_hbm, agg_out,
              src_v, dst_v, rows_v, agg_sh, sg, ss):
    c = lax.axis_index("c")
    s = lax.axis_index("s")
    pltpu.sync_copy(src_hbm.at[s], src_v)
    pltpu.sync_copy(dst_hbm.at[s], dst_v)
    r0 = s * RPS
    pltpu.sync_copy(zeros_hbm.at[pl.ds(r0, RPS)], agg_sh.at[pl.ds(r0, RPS)])
    plsc.subcore_barrier()

    xs_c = xs_hbm.at[c]
    for b in range(NB):
        pltpu.async_copy(xs_c.at[src_v.at[b]], rows_v.at[b], sg[b])

    def body(i, carry):
        for b in range(NB):
            j = i * NB + b
            pltpu.make_async_copy(xs_c.at[src_v.at[j]], rows_v.at[b], sg[b]).wait()
            pltpu.async_copy(rows_v.at[b], agg_sh.at[dst_v.at[j]], ss[b], add=True)
            nxt = j + NB

            @pl.when(nxt < KCH2)
            def _():
                pltpu.make_async_copy(rows_v.at[b], agg_sh.at[dst_v.at[j]], ss[b]).wait()
                pltpu.async_copy(xs_c.at[src_v.at[nxt]], rows_v.at[b], sg[b])
        return carry

    lax.fori_loop(0, KCH2 // NB, body, 0)
    for b in range(NB):
        pltpu.make_async_copy(rows_v.at[b], agg_sh.at[dst_v.at[KCH2 - NB + b]], ss[b]).wait()
    plsc.subcore_barrier()
    pltpu.sync_copy(agg_sh.at[pl.ds(r0, RPS)], agg_out.at[c, pl.ds(r0, RPS)])


def _agg_types(interpret=False):
    return dict(
        out_type=jax.ShapeDtypeStruct((NC, NPAD, H), jnp.float32),
        mesh=_mesh,
        scratch_types=[
            pltpu.VMEM((KCH2, BCH), jnp.int32),
            pltpu.VMEM((KCH2, BCH), jnp.int32),
            pltpu.VMEM((NB, BCH, H), jnp.float32),
            pltpu.VMEM_SHARED((NPAD, H), jnp.float32),
            [pltpu.SemaphoreType.DMA] * NB,
            [pltpu.SemaphoreType.DMA] * NB,
        ],
        compiler_params=pltpu.CompilerParams(use_tc_tiling_on_sc=False),
        interpret=interpret,
    )


_agg_kernel = pl.kernel(_agg_body, **_agg_types())


_BLK = 2000


def _dis_block(d_ref):
    deg = (d_ref[0] + d_ref[1])[:, 0:1] + 1.0
    return jnp.broadcast_to(lax.rsqrt(deg), (_BLK, D))


def _scale_body(d_ref, x_ref, xs_ref):
    xs = x_ref[...] * _dis_block(d_ref)
    xs_ref[0] = xs[:, :H]
    xs_ref[1] = xs[:, H:]


def _scale_kernel(deg2, x):
    return pl.pallas_call(
        _scale_body,
        grid=(N // _BLK,),
        in_specs=[
            pl.BlockSpec((NC, _BLK, 16), lambda i: (0, i, 0)),
            pl.BlockSpec((_BLK, D), lambda i: (i, 0)),
        ],
        out_specs=pl.BlockSpec((NC, _BLK, H), lambda i: (0, i, 0)),
        out_shape=jax.ShapeDtypeStruct((NC, N, H), jnp.float32),
    )(deg2, x)


def _out_body(a_ref, xs_ref, d_ref, wm_ref, wl_ref,
              bm_ref, bl_ref, mu_ref, ls_ref):
    agg = jnp.concatenate([a_ref[0] + xs_ref[0], a_ref[1] + xs_ref[1]], axis=1)
    full = _dis_block(d_ref) * agg
    mu_ref[...] = (
        jnp.dot(full, wm_ref[...], preferred_element_type=jnp.float32)
        + bm_ref[0:1, :]
    )
    ls_ref[...] = (
        jnp.dot(full, wl_ref[...], preferred_element_type=jnp.float32)
        + bl_ref[0:1, :]
    )


def _out_kernel(agg2, xs2, deg2, W_mu, W_ls, bm8, bl8):
    halves = pl.BlockSpec((NC, _BLK, H), lambda i: (0, i, 0))
    deg = pl.BlockSpec((NC, _BLK, 16), lambda i: (0, i, 0))
    row = pl.BlockSpec((_BLK, D), lambda i: (i, 0))
    whole = pl.BlockSpec((D, D), lambda i: (0, 0))
    bias = pl.BlockSpec((8, D), lambda i: (0, 0))
    return pl.pallas_call(
        _out_body,
        grid=(N // _BLK,),
        in_specs=[halves, halves, deg, whole, whole, bias, bias],
        out_specs=[row, row],
        out_shape=[
            jax.ShapeDtypeStruct((N, D), jnp.float32),
            jax.ShapeDtypeStruct((N, D), jnp.float32),
        ],
    )(agg2, xs2, deg2, W_mu, W_ls, bm8, bl8)


def kernel(x, edge_index, W_mu, b_mu, W_logstd, b_logstd):
    dst_r1 = edge_index[1].reshape(NW, KCH1, BCH)
    src_r2 = edge_index[0].reshape(NS, KCH2, BCH)
    dst_r2 = edge_index[1].reshape(NS, KCH2, BCH)
    ones16 = jnp.ones((BCH, 16), jnp.float32)
    zeros16 = jnp.zeros((NPAD, 16), jnp.float32)
    zeros64 = jnp.zeros((NPAD, H), jnp.float32)

    deg2 = _deg_kernel(dst_r1, ones16, zeros16)
    xs2 = _scale_kernel(deg2, x)
    agg2 = _agg_kernel(src_r2, dst_r2, xs2, zeros64)
    mu, logstd = _out_kernel(
        agg2, xs2, deg2, W_mu, W_logstd,
        jnp.broadcast_to(b_mu[None, :], (8, D)),
        jnp.broadcast_to(b_logstd[None, :], (8, D)),
    )
    return (mu, logstd)


# speedup vs baseline: 1.1658x; 1.0117x over previous
import functools

import jax
import jax.numpy as jnp
from jax import lax
from jax.experimental import pallas as pl
from jax.experimental.pallas import tpu as pltpu
from jax.experimental.pallas import tpu_sc as plsc

N = 10000
E = 320000
D = 128
H = D // 2

NC = 2
NS = 16
NW = NC * NS
BCH = 125
KCH1 = E // NW // BCH
KCH2 = E // NS // BCH
NB = 4
NPAD = 10112
RPS = NPAD // NS

_mesh = plsc.VectorSubcoreMesh(core_axis_name="c", subcore_axis_name="s")


def _deg_body(dst_hbm, ones_hbm, zeros_hbm, deg_out, dst_v, ones_v, deg_sh, sem):
    c = lax.axis_index("c")
    s = lax.axis_index("s")
    wid = s * NC + c
    pltpu.sync_copy(dst_hbm.at[wid], dst_v)
    pltpu.sync_copy(ones_hbm, ones_v)
    r0 = s * RPS
    pltpu.sync_copy(zeros_hbm.at[pl.ds(r0, RPS)], deg_sh.at[pl.ds(r0, RPS)])
    plsc.subcore_barrier()

    FK = 10

    def body(i, carry):
        for b in range(FK):
            pltpu.async_copy(ones_v, deg_sh.at[dst_v.at[i * FK + b]], sem, add=True)
        for b in range(FK):
            pltpu.make_async_copy(ones_v, deg_sh.at[dst_v.at[i * FK + b]], sem).wait()
        return carry

    lax.fori_loop(0, KCH1 // FK, body, 0)
    plsc.subcore_barrier()
    pltpu.sync_copy(deg_sh.at[pl.ds(r0, RPS)], deg_out.at[c, pl.ds(r0, RPS)])


def _deg_types(interpret=False):
    return dict(
        out_type=jax.ShapeDtypeStruct((NC, NPAD, 16), jnp.float32),
        mesh=_mesh,
        scratch_types=[
            pltpu.VMEM((KCH1, BCH), jnp.int32),
            pltpu.VMEM((BCH, 16), jnp.float32),
            pltpu.VMEM_SHARED((NPAD, 16), jnp.float32),
            pltpu.SemaphoreType.DMA,
        ],
        compiler_params=pltpu.CompilerParams(use_tc_tiling_on_sc=False),
        interpret=interpret,
    )


_deg_kernel = pl.kernel(_deg_body, **_deg_types())


def _agg_body(src_hbm, dst_hbm, xs_hbm, zeros_hbm, agg_out,
              src_v, dst_v, rows_v, agg_sh, sg, ss):
    c = lax.axis_index("c")
    s = lax.axis_index("s")
    pltpu.sync_copy(src_hbm.at[s], src_v)
    pltpu.sync_copy(dst_hbm.at[s], dst_v)
    r0 = s * RPS
    pltpu.sync_copy(zeros_hbm.at[pl.ds(r0, RPS)], agg_sh.at[pl.ds(r0, RPS)])
    plsc.subcore_barrier()

    xs_c = xs_hbm.at[c]
    for b in range(NB):
        pltpu.async_copy(xs_c.at[src_v.at[b]], rows_v.at[b], sg[b])

    def body(i, carry):
        for b in range(NB):
            j = i * NB + b
            pltpu.make_async_copy(xs_c.at[src_v.at[j]], rows_v.at[b], sg[b]).wait()
            pltpu.async_copy(rows_v.at[b], agg_sh.at[dst_v.at[j]], ss[b], add=True)
            nxt = j + NB

            @pl.when(nxt < KCH2)
            def _():
                pltpu.make_async_copy(rows_v.at[b], agg_sh.at[dst_v.at[j]], ss[b]).wait()
                pltpu.async_copy(xs_c.at[src_v.at[nxt]], rows_v.at[b], sg[b])
        return carry

    lax.fori_loop(0, KCH2 // NB, body, 0)
    for b in range(NB):
        pltpu.make_async_copy(rows_v.at[b], agg_sh.at[dst_v.at[KCH2 - NB + b]], ss[b]).wait()
    plsc.subcore_barrier()
    pltpu.sync_copy(agg_sh.at[pl.ds(r0, RPS)], agg_out.at[c, pl.ds(r0, RPS)])


def _agg_types(interpret=False):
    return dict(
        out_type=jax.ShapeDtypeStruct((NC, NPAD, H), jnp.float32),
        mesh=_mesh,
        scratch_types=[
            pltpu.VMEM((KCH2, BCH), jnp.int32),
            pltpu.VMEM((KCH2, BCH), jnp.int32),
            pltpu.VMEM((NB, BCH, H), jnp.float32),
            pltpu.VMEM_SHARED((NPAD, H), jnp.float32),
            [pltpu.SemaphoreType.DMA] * NB,
            [pltpu.SemaphoreType.DMA] * NB,
        ],
        compiler_params=pltpu.CompilerParams(use_tc_tiling_on_sc=False),
        interpret=interpret,
    )


_agg_kernel = pl.kernel(_agg_body, **_agg_types())


_BLK = 5000


def _dis_block(d_ref):
    deg = (d_ref[0] + d_ref[1])[:, 0:1] + 1.0
    return jnp.broadcast_to(lax.rsqrt(deg), (_BLK, D))


def _scale_body(d_ref, x_ref, xs_ref):
    xs = x_ref[...] * _dis_block(d_ref)
    xs_ref[0] = xs[:, :H]
    xs_ref[1] = xs[:, H:]


def _scale_kernel(deg2, x):
    return pl.pallas_call(
        _scale_body,
        grid=(N // _BLK,),
        in_specs=[
            pl.BlockSpec((NC, _BLK, 16), lambda i: (0, i, 0)),
            pl.BlockSpec((_BLK, D), lambda i: (i, 0)),
        ],
        out_specs=pl.BlockSpec((NC, _BLK, H), lambda i: (0, i, 0)),
        out_shape=jax.ShapeDtypeStruct((NC, N, H), jnp.float32),
    )(deg2, x)


def _out_body(a_ref, xs_ref, d_ref, wm_ref, wl_ref,
              bm_ref, bl_ref, mu_ref, ls_ref):
    agg = jnp.concatenate([a_ref[0] + xs_ref[0], a_ref[1] + xs_ref[1]], axis=1)
    full = _dis_block(d_ref) * agg
    mu_ref[...] = (
        jnp.dot(full, wm_ref[...], preferred_element_type=jnp.float32)
        + bm_ref[0:1, :]
    )
    ls_ref[...] = (
        jnp.dot(full, wl_ref[...], preferred_element_type=jnp.float32)
        + bl_ref[0:1, :]
    )


def _out_kernel(agg2, xs2, deg2, W_mu, W_ls, bm8, bl8):
    halves = pl.BlockSpec((NC, _BLK, H), lambda i: (0, i, 0))
    deg = pl.BlockSpec((NC, _BLK, 16), lambda i: (0, i, 0))
    row = pl.BlockSpec((_BLK, D), lambda i: (i, 0))
    whole = pl.BlockSpec((D, D), lambda i: (0, 0))
    bias = pl.BlockSpec((8, D), lambda i: (0, 0))
    return pl.pallas_call(
        _out_body,
        grid=(N // _BLK,),
        in_specs=[halves, halves, deg, whole, whole, bias, bias],
        out_specs=[row, row],
        out_shape=[
            jax.ShapeDtypeStruct((N, D), jnp.float32),
            jax.ShapeDtypeStruct((N, D), jnp.float32),
        ],
    )(agg2, xs2, deg2, W_mu, W_ls, bm8, bl8)


def kernel(x, edge_index, W_mu, b_mu, W_logstd, b_logstd):
    dst_r1 = edge_index[1].reshape(NW, KCH1, BCH)
    src_r2 = edge_index[0].reshape(NS, KCH2, BCH)
    dst_r2 = edge_index[1].reshape(NS, KCH2, BCH)
    ones16 = jnp.ones((BCH, 16), jnp.float32)
    zeros16 = jnp.zeros((NPAD, 16), jnp.float32)
    zeros64 = jnp.zeros((NPAD, H), jnp.float32)

    deg2 = _deg_kernel(dst_r1, ones16, zeros16)
    xs2 = _scale_kernel(deg2, x)
    agg2 = _agg_kernel(src_r2, dst_r2, xs2, zeros64)
    mu, logstd = _out_kernel(
        agg2, xs2, deg2, W_mu, W_logstd,
        jnp.broadcast_to(b_mu[None, :], (8, D)),
        jnp.broadcast_to(b_logstd[None, :], (8, D)),
    )
    return (mu, logstd)
